# baseline (speedup 1.0000x reference)
import functools

import jax
import jax.numpy as jnp
from jax import lax
from jax.experimental import pallas as pl
from jax.experimental.pallas import tpu as pltpu
from jax.experimental.pallas import tpu_sc as plsc

B = 1024
S = 77
SP = 80
H = SP // 2
D = 768
NC, NS = 2, 16
NW = NC * NS
SEQ_PER_W = B // NW
BB = 8

_mesh = plsc.VectorSubcoreMesh(core_axis_name="c", subcore_axis_name="s")


@functools.partial(
    pl.kernel,
    mesh=_mesh,
    out_type=(
        jax.ShapeDtypeStruct((B * SP, D), jnp.float32),
        jax.ShapeDtypeStruct((SP, D), jnp.float32),
    ),
    scratch_types=[
        pltpu.VMEM((SEQ_PER_W * SP,), jnp.int32),
        pltpu.VMEM((SP,), jnp.int32),
        pltpu.VMEM((SP, D), jnp.float32),
        pltpu.VMEM((H, D), jnp.float32),
        pltpu.VMEM((H, D), jnp.float32),
        pltpu.SemaphoreType.DMA,
        pltpu.SemaphoreType.DMA,
        pltpu.SemaphoreType.DMA,
        pltpu.SemaphoreType.DMA,
    ],
)
def _gather_kernel(ids_hbm, pids_hbm, tok_hbm, pos_hbm, out_hbm, pos_out_hbm,
                   idx_v, pidx_v, pos_v, buf0, buf1,
                   gsem0, gsem1, osem0, osem1):
    wid = lax.axis_index("s") * NC + lax.axis_index("c")
    wbase = wid * (SEQ_PER_W * SP)

    pltpu.sync_copy(ids_hbm.at[pl.ds(wbase, SEQ_PER_W * SP)], idx_v)
    pltpu.sync_copy(pids_hbm, pidx_v)
    pltpu.async_copy(pos_hbm.at[pidx_v], pos_v, gsem0).wait()

    @pl.when(wid == 0)
    def _():
        pltpu.sync_copy(pos_v, pos_out_hbm)

    def seq_body(q, _):
        base = wbase + q * SP

        @pl.when(q > 0)
        def _():
            pltpu.make_async_copy(
                buf0, out_hbm.at[pl.ds(base - SP, H)], osem0).wait()
        pltpu.async_copy(tok_hbm.at[idx_v.at[pl.ds(q * SP, H)]],
                         buf0, gsem0).wait()
        pltpu.async_copy(buf0, out_hbm.at[pl.ds(base, H)], osem0)

        @pl.when(q > 0)
        def _():
            pltpu.make_async_copy(
                buf1, out_hbm.at[pl.ds(base - SP + H, H)], osem1).wait()
        pltpu.async_copy(tok_hbm.at[idx_v.at[pl.ds(q * SP + H, H)]],
                         buf1, gsem1).wait()
        pltpu.async_copy(buf1, out_hbm.at[pl.ds(base + H, H)], osem1)
        return 0

    lax.fori_loop(0, SEQ_PER_W, seq_body, 0)

    last = wbase + (SEQ_PER_W - 1) * SP
    pltpu.make_async_copy(buf0, out_hbm.at[pl.ds(last, H)], osem0).wait()
    pltpu.make_async_copy(buf1, out_hbm.at[pl.ds(last + H, H)], osem1).wait()


def _add_body(g_ref, p_ref, o_ref):
    g = g_ref[...].reshape(BB, SP, D)
    p = p_ref[...]
    o_ref[...] = g[:, :S, :] + p[None, :S, :]


_add_kernel = pl.pallas_call(
    _add_body,
    grid=(B // BB,),
    in_specs=[
        pl.BlockSpec((BB * SP, D), lambda i: (i, 0)),
        pl.BlockSpec((SP, D), lambda i: (0, 0)),
    ],
    out_specs=pl.BlockSpec((BB, S, D), lambda i: (i, 0, 0)),
    out_shape=jax.ShapeDtypeStruct((B, S, D), jnp.float32),
)


def kernel(input_ids, position_ids, token_table, pos_table):
    ids = input_ids.astype(jnp.int32).reshape(B, S)
    ids_pad = jnp.pad(ids, ((0, 0), (0, SP - S))).reshape(-1)
    pids = jnp.pad(position_ids.astype(jnp.int32).reshape(-1), (0, SP - S))
    gathered, pos_eff = _gather_kernel(ids_pad, pids, token_table, pos_table)
    return _add_kernel(gathered, pos_eff)


# speedup vs baseline: 1.2945x; 1.2945x over previous
import functools

import jax
import jax.numpy as jnp
from jax import lax
from jax.experimental import pallas as pl
from jax.experimental.pallas import tpu as pltpu
from jax.experimental.pallas import tpu_sc as plsc

B = 1024
S = 77
SP = 80
H = SP // 2
D = 768
NC, NS = 2, 16
NW = NC * NS
SEQ_PER_W = B // NW
BB = 8

_mesh = plsc.VectorSubcoreMesh(core_axis_name="c", subcore_axis_name="s")


@functools.partial(
    pl.kernel,
    mesh=_mesh,
    out_type=(
        jax.ShapeDtypeStruct((B * SP, D), jnp.float32),
        jax.ShapeDtypeStruct((SP, D), jnp.float32),
    ),
    scratch_types=[
        pltpu.VMEM((SEQ_PER_W * SP,), jnp.int32),
        pltpu.VMEM((SP,), jnp.int32),
        pltpu.VMEM((SP, D), jnp.float32),
        pltpu.VMEM((H, D), jnp.float32),
        pltpu.VMEM((H, D), jnp.float32),
        pltpu.SemaphoreType.DMA,
        pltpu.SemaphoreType.DMA,
        pltpu.SemaphoreType.DMA,
        pltpu.SemaphoreType.DMA,
    ],
)
def _gather_kernel(ids_hbm, pids_hbm, tok_hbm, pos_hbm, out_hbm, pos_out_hbm,
                   idx_v, pidx_v, pos_v, buf0, buf1,
                   gsem0, gsem1, osem0, osem1):
    wid = lax.axis_index("s") * NC + lax.axis_index("c")
    wbase = wid * (SEQ_PER_W * SP)

    pltpu.sync_copy(ids_hbm.at[pl.ds(wbase, SEQ_PER_W * SP)], idx_v)
    pltpu.sync_copy(pids_hbm, pidx_v)
    pltpu.async_copy(pos_hbm.at[pidx_v], pos_v, gsem0).wait()

    @pl.when(wid == 0)
    def _():
        pltpu.sync_copy(pos_v, pos_out_hbm)

    def seq_body(q, _):
        base = wbase + q * SP

        @pl.when(q > 0)
        def _():
            pltpu.make_async_copy(
                buf0, out_hbm.at[pl.ds(base - SP, H)], osem0).wait()
        pltpu.async_copy(tok_hbm.at[idx_v.at[pl.ds(q * SP, H)]],
                         buf0, gsem0).wait()
        pltpu.async_copy(buf0, out_hbm.at[pl.ds(base, H)], osem0)

        @pl.when(q > 0)
        def _():
            pltpu.make_async_copy(
                buf1, out_hbm.at[pl.ds(base - SP + H, H)], osem1).wait()
        pltpu.async_copy(tok_hbm.at[idx_v.at[pl.ds(q * SP + H, H)]],
                         buf1, gsem1).wait()
        pltpu.async_copy(buf1, out_hbm.at[pl.ds(base + H, H)], osem1)
        return 0

    lax.fori_loop(0, SEQ_PER_W, seq_body, 0)

    last = wbase + (SEQ_PER_W - 1) * SP
    pltpu.make_async_copy(buf0, out_hbm.at[pl.ds(last, H)], osem0).wait()
    pltpu.make_async_copy(buf1, out_hbm.at[pl.ds(last + H, H)], osem1).wait()


def _add_body(g_ref, p_ref, o_ref):
    g = g_ref[...].reshape(BB, SP, D)
    p = p_ref[...]
    o_ref[...] = g[:, :S, :] + p[None, :S, :]


_add_kernel = pl.pallas_call(
    _add_body,
    grid=(B // BB,),
    in_specs=[
        pl.BlockSpec((BB * SP, D), lambda i: (i, 0)),
        pl.BlockSpec((SP, D), lambda i: (0, 0)),
    ],
    out_specs=pl.BlockSpec((BB, S, D), lambda i: (i, 0, 0)),
    out_shape=jax.ShapeDtypeStruct((B, S, D), jnp.float32),
)


def kernel(input_ids, position_ids, token_table, pos_table):
    ids = input_ids.astype(jnp.int32).reshape(B, S)
    ids_pad = jnp.pad(ids, ((0, 0), (0, SP - S))).reshape(-1)
    pids = jnp.pad(position_ids.astype(jnp.int32).reshape(-1), (0, SP - S))
    gathered, pos_eff = _gather_kernel(ids_pad, pids, token_table, pos_table)
    g3 = gathered.reshape(B, SP, D)
    return g3[:, :S, :] + pos_eff[None, :S, :]


# speedup vs baseline: 1.2982x; 1.0029x over previous
import functools

import jax
import jax.numpy as jnp
from jax import lax
from jax.experimental import pallas as pl
from jax.experimental.pallas import tpu as pltpu
from jax.experimental.pallas import tpu_sc as plsc

B = 1024
S = 77
SP = 80
H = SP // 2
D = 768
NC, NS = 2, 16
NW = NC * NS
SEQ_PER_W = B // NW
CH = 32
NCH = SEQ_PER_W * SP // CH
BB = 8

_mesh = plsc.VectorSubcoreMesh(core_axis_name="c", subcore_axis_name="s")


@functools.partial(
    pl.kernel,
    mesh=_mesh,
    out_type=(
        jax.ShapeDtypeStruct((B * SP, D), jnp.float32),
        jax.ShapeDtypeStruct((SP, D), jnp.float32),
    ),
    scratch_types=[
        pltpu.VMEM((SEQ_PER_W * SP,), jnp.int32),
        pltpu.VMEM((SP,), jnp.int32),
        pltpu.VMEM((SP, D), jnp.float32),
        pltpu.VMEM((CH, D), jnp.float32),
        pltpu.VMEM((CH, D), jnp.float32),
        pltpu.SemaphoreType.DMA,
        pltpu.SemaphoreType.DMA,
        pltpu.SemaphoreType.DMA,
        pltpu.SemaphoreType.DMA,
    ],
)
def _gather_kernel(ids_hbm, pids_hbm, tok_hbm, pos_hbm, out_hbm, pos_out_hbm,
                   idx_v, pidx_v, pos_v, buf0, buf1,
                   gsem0, gsem1, osem0, osem1):
    wid = lax.axis_index("s") * NC + lax.axis_index("c")
    wbase = wid * (SEQ_PER_W * SP)

    pltpu.sync_copy(ids_hbm.at[pl.ds(wbase, SEQ_PER_W * SP)], idx_v)
    pltpu.sync_copy(pids_hbm, pidx_v)
    pltpu.async_copy(pos_hbm.at[pidx_v], pos_v, gsem0).wait()

    @pl.when(wid == 0)
    def _():
        pltpu.sync_copy(pos_v, pos_out_hbm)

    bufs = (buf0, buf1)
    osems = (osem0, osem1)
    gsems = (gsem0, gsem1)

    def chunk_body(c, _):
        off = c * CH
        base = wbase + off
        parity = lax.rem(c, 2)

        def run(b):
            @pl.when(c >= 2)
            def _():
                pltpu.make_async_copy(
                    bufs[b], out_hbm.at[pl.ds(base - 2 * CH, CH)],
                    osems[b]).wait()
            pltpu.async_copy(tok_hbm.at[idx_v.at[pl.ds(off, CH)]],
                             bufs[b], gsems[b]).wait()
            pltpu.async_copy(bufs[b], out_hbm.at[pl.ds(base, CH)], osems[b])

        @pl.when(parity == 0)
        def _():
            run(0)

        @pl.when(parity == 1)
        def _():
            run(1)
        return 0

    lax.fori_loop(0, NCH, chunk_body, 0)

    for c in (NCH - 2, NCH - 1):
        b = c % 2
        pltpu.make_async_copy(
            bufs[b], out_hbm.at[pl.ds(wbase + c * CH, CH)], osems[b]).wait()


def _add_body(g_ref, p_ref, o_ref):
    g = g_ref[...].reshape(BB, SP, D)
    p = p_ref[...]
    o_ref[...] = g[:, :S, :] + p[None, :S, :]


_add_kernel = pl.pallas_call(
    _add_body,
    grid=(B // BB,),
    in_specs=[
        pl.BlockSpec((BB * SP, D), lambda i: (i, 0)),
        pl.BlockSpec((SP, D), lambda i: (0, 0)),
    ],
    out_specs=pl.BlockSpec((BB, S, D), lambda i: (i, 0, 0)),
    out_shape=jax.ShapeDtypeStruct((B, S, D), jnp.float32),
)


def kernel(input_ids, position_ids, token_table, pos_table):
    ids = input_ids.astype(jnp.int32).reshape(B, S)
    ids_pad = jnp.pad(ids, ((0, 0), (0, SP - S))).reshape(-1)
    pids = jnp.pad(position_ids.astype(jnp.int32).reshape(-1), (0, SP - S))
    gathered, pos_eff = _gather_kernel(ids_pad, pids, token_table, pos_table)
    g3 = gathered.reshape(B, SP, D)
    return g3[:, :S, :] + pos_eff[None, :S, :]


# speedup vs baseline: 1.4430x; 1.1115x over previous
import functools

import jax
import jax.numpy as jnp
from jax import lax
from jax.experimental import pallas as pl
from jax.experimental.pallas import tpu as pltpu
from jax.experimental.pallas import tpu_sc as plsc

B = 1024
S = 77
SP = 80
D = 768
V = 49408
NC, NS = 2, 16
NW = NC * NS
SEQ_PER_W = B // NW
NCH = SEQ_PER_W
BB = 8

_mesh = plsc.VectorSubcoreMesh(core_axis_name="c", subcore_axis_name="s")


@functools.partial(
    pl.kernel,
    mesh=_mesh,
    out_type=(
        jax.ShapeDtypeStruct((B * SP, D), jnp.float32),
        jax.ShapeDtypeStruct((SP, D), jnp.float32),
    ),
    scratch_types=[
        pltpu.VMEM((SEQ_PER_W * SP,), jnp.int32),
        pltpu.VMEM((SP,), jnp.int32),
        pltpu.VMEM((SP, D), jnp.float32),
        pltpu.VMEM((SP, D), jnp.float32),
        pltpu.SemaphoreType.DMA,
        pltpu.SemaphoreType.DMA,
        pltpu.SemaphoreType.DMA,
        pltpu.SemaphoreType.DMA,
    ],
)
def _gather_kernel(ids_hbm, pids_hbm, tok_hbm, pos_hbm, out_hbm, pos_out_hbm,
                   idx_v, pidx_v, buf0, buf1, gsem0, gsem1, osem0, osem1):
    wid = lax.axis_index("s") * NC + lax.axis_index("c")
    wbase = wid * (SEQ_PER_W * SP)

    pltpu.sync_copy(ids_hbm.at[pl.ds(wbase, SEQ_PER_W * SP)], idx_v)

    @pl.when(wid == 0)
    def _():
        pltpu.sync_copy(pids_hbm, pidx_v)
        pltpu.async_copy(pos_hbm.at[pidx_v], buf0, gsem0).wait()
        pltpu.sync_copy(buf0, pos_out_hbm)

    bufs = (buf0, buf1)
    gsems = (gsem0, gsem1)
    osems = (osem0, osem1)

    def gwait(b, c):
        pltpu.make_async_copy(tok_hbm.at[idx_v.at[pl.ds(c * SP, SP)]],
                              bufs[b], gsems[b]).wait()

    def owait(b, c):
        pltpu.make_async_copy(bufs[b], out_hbm.at[pl.ds(wbase + c * SP, SP)],
                              osems[b]).wait()

    def step(b, c):
        @pl.when(c >= 2)
        def _():
            owait(b, c - 2)
        pltpu.async_copy(tok_hbm.at[idx_v.at[pl.ds(c * SP, SP)]],
                         bufs[b], gsems[b])
        ob = 1 - b

        @pl.when(c >= 1)
        def _():
            gwait(ob, c - 1)
            pltpu.async_copy(bufs[ob],
                             out_hbm.at[pl.ds(wbase + (c - 1) * SP, SP)],
                             osems[ob])

    def chunk_body(c, _):
        parity = lax.rem(c, 2)

        @pl.when(parity == 0)
        def _():
            step(0, c)

        @pl.when(parity == 1)
        def _():
            step(1, c)
        return 0

    lax.fori_loop(0, NCH, chunk_body, 0)

    lb = (NCH - 1) % 2
    gwait(lb, NCH - 1)
    pltpu.async_copy(bufs[lb], out_hbm.at[pl.ds(wbase + (NCH - 1) * SP, SP)],
                     osems[lb])
    owait(1 - lb, NCH - 2)
    owait(lb, NCH - 1)


def _add_body(g_ref, p_ref, o_ref):
    g = g_ref[...].reshape(BB, SP, D)
    p = p_ref[...]
    o_ref[...] = g[:, :S, :] + p[None, :S, :]


_add_kernel = pl.pallas_call(
    _add_body,
    grid=(B // BB,),
    in_specs=[
        pl.BlockSpec((BB * SP, D), lambda i: (i, 0)),
        pl.BlockSpec((SP, D), lambda i: (0, 0)),
    ],
    out_specs=pl.BlockSpec((BB, S, D), lambda i: (i, 0, 0)),
    out_shape=jax.ShapeDtypeStruct((B, S, D), jnp.float32),
)


def kernel(input_ids, position_ids, token_table, pos_table):
    ids = input_ids.astype(jnp.int32).reshape(B, S)
    padv = (jnp.arange(B * (SP - S), dtype=jnp.int32) * 997 % V).reshape(
        B, SP - S)
    ids_pad = jnp.concatenate([ids, padv], axis=1).reshape(-1)
    pids = jnp.pad(position_ids.astype(jnp.int32).reshape(-1), (0, SP - S))
    gathered, pos_eff = _gather_kernel(ids_pad, pids, token_table, pos_table)
    return _add_kernel(gathered, pos_eff)


# speedup vs baseline: 2.2148x; 1.5349x over previous
import functools

import jax
import jax.numpy as jnp
from jax import lax
from jax.experimental import pallas as pl
from jax.experimental.pallas import tpu as pltpu
from jax.experimental.pallas import tpu_sc as plsc

B = 1024
S = 77
SP = 80
D = 768
NC, NS = 2, 16
NW = NC * NS
CHB = 32
ITEMS = S * (B // CHB)
IPW = ITEMS // NW
ROWS_PER_W = IPW * CHB

_mesh = plsc.VectorSubcoreMesh(core_axis_name="c", subcore_axis_name="s")


@functools.partial(
    pl.kernel,
    mesh=_mesh,
    out_type=(
        jax.ShapeDtypeStruct((S * B, D), jnp.float32),
        jax.ShapeDtypeStruct((SP, D), jnp.float32),
    ),
    scratch_types=[
        pltpu.VMEM((ROWS_PER_W,), jnp.int32),
        pltpu.VMEM((SP,), jnp.int32),
        pltpu.VMEM((CHB, D), jnp.float32),
        pltpu.VMEM((CHB, D), jnp.float32),
        pltpu.VMEM((SP, D), jnp.float32),
        pltpu.SemaphoreType.DMA,
        pltpu.SemaphoreType.DMA,
        pltpu.SemaphoreType.DMA,
        pltpu.SemaphoreType.DMA,
    ],
)
def _gather_kernel(ids_hbm, pids_hbm, tok_hbm, pos_hbm, out_hbm, pos_out_hbm,
                   idx_v, pidx_v, buf0, buf1, pbuf,
                   gsem0, gsem1, osem0, osem1):
    wid = lax.axis_index("s") * NC + lax.axis_index("c")
    item0 = wid * IPW

    pltpu.sync_copy(ids_hbm.at[pl.ds(item0 * CHB, ROWS_PER_W)], idx_v)

    @pl.when(wid == 0)
    def _():
        pltpu.sync_copy(pids_hbm, pidx_v)
        pltpu.async_copy(pos_hbm.at[pidx_v], pbuf, gsem0).wait()
        pltpu.sync_copy(pbuf, pos_out_hbm)

    bufs = (buf0, buf1)
    gsems = (gsem0, gsem1)
    osems = (osem0, osem1)

    def obase(i):
        k = item0 + i
        return (k // (B // CHB)) * B + lax.rem(k, B // CHB) * CHB

    def gwait(b, i):
        pltpu.make_async_copy(tok_hbm.at[idx_v.at[pl.ds(i * CHB, CHB)]],
                              bufs[b], gsems[b]).wait()

    def owait(b, i):
        pltpu.make_async_copy(bufs[b], out_hbm.at[pl.ds(obase(i), CHB)],
                              osems[b]).wait()

    def step(b, i):
        @pl.when(i >= 2)
        def _():
            owait(b, i - 2)
        pltpu.async_copy(tok_hbm.at[idx_v.at[pl.ds(i * CHB, CHB)]],
                         bufs[b], gsems[b])
        ob = 1 - b

        @pl.when(i >= 1)
        def _():
            gwait(ob, i - 1)
            pltpu.async_copy(bufs[ob], out_hbm.at[pl.ds(obase(i - 1), CHB)],
                             osems[ob])

    def item_body(i, _):
        parity = lax.rem(i, 2)

        @pl.when(parity == 0)
        def _():
            step(0, i)

        @pl.when(parity == 1)
        def _():
            step(1, i)
        return 0

    lax.fori_loop(0, IPW, item_body, 0)

    lb = (IPW - 1) % 2
    gwait(lb, IPW - 1)
    pltpu.async_copy(bufs[lb], out_hbm.at[pl.ds(obase(IPW - 1), CHB)],
                     osems[lb])
    owait(1 - lb, IPW - 2)
    owait(lb, IPW - 1)


def _add_body(g_ref, p_ref, o_ref):
    p = p_ref[pl.program_id(0)]
    o_ref[...] = (g_ref[...] + p[None, :])[None]


_add_kernel = pl.pallas_call(
    _add_body,
    grid=(S,),
    in_specs=[
        pl.BlockSpec((B, D), lambda s: (s, 0)),
        pl.BlockSpec((SP, D), lambda s: (0, 0)),
    ],
    out_specs=pl.BlockSpec((1, B, D), lambda s: (s, 0, 0)),
    out_shape=jax.ShapeDtypeStruct((S, B, D), jnp.float32),
)


def kernel(input_ids, position_ids, token_table, pos_table):
    ids_sm = input_ids.astype(jnp.int32).reshape(B, S).T.reshape(-1)
    pids = jnp.pad(position_ids.astype(jnp.int32).reshape(-1), (0, SP - S))
    gathered, pos_eff = _gather_kernel(ids_sm, pids, token_table, pos_table)
    out_sm = _add_kernel(gathered, pos_eff)
    return jnp.transpose(out_sm, (1, 0, 2))
